# baseline (speedup 1.0000x reference)
import jax
import jax.numpy as jnp
from jax import lax
from jax.experimental import pallas as pl
from jax.experimental.pallas import tpu as pltpu
from jax.experimental.pallas import tpu_sc as plsc

N = 10000
D = 128
NP = 10240
CHUNK = 128
EPAD = 327680
NSUB = 16
RPT = NP // NSUB
BR = 1280
GRID = NP // BR


def _make_seg_kernel(mode):
    ept = EPAD // 32 if mode == "split" else EPAD // 16
    nch = ept // CHUNK
    mesh = plsc.VectorSubcoreMesh(core_axis_name="c", subcore_axis_name="s")

    def body(table, src, dst, zacc, zhist, out_s, out_h,
             idx0, idx1, dv0, dv1, ds0, ds1, rv0, rv1, hist_v, acc_sh,
             si0, si1, sg0, sg1, ss0, ss1):
        idx_v = (idx0, idx1)
        dst_v = (dv0, dv1)
        dst_s = (ds0, ds1)
        rows_v = (rv0, rv1)
        sem_i = (si0, si1)
        sem_g = (sg0, sg1)
        sem_s = (ss0, ss1)
        c = lax.axis_index("c")
        s = lax.axis_index("s")
        w = c * NSUB + s
        r0 = s * RPT
        nblk = RPT // CHUNK
        pltpu.sync_copy(zacc, rv0)
        for j in range(nblk):
            pltpu.sync_copy(rv0, acc_sh.at[pl.ds(r0 + j * CHUNK, CHUNK)])
        pltpu.sync_copy(zhist, hist_v)
        plsc.subcore_barrier()

        if mode == "split":
            ebase = (c * NSUB + s) * ept
        else:
            ebase = s * ept

        ones16 = jnp.ones((16,), jnp.float32)
        H = CHUNK // 2

        def scat_halves(b):
            return (
                (rows_v[b].at[pl.ds(0, H)], acc_sh.at[dst_s[b].at[pl.ds(0, H)]],
                 sem_s[b]),
                (rows_v[b].at[pl.ds(H, H)], acc_sh.at[dst_s[b].at[pl.ds(H, H)]],
                 sem_s[b]),
            )

        def issue_scat(b):
            ha, hb = scat_halves(b)
            pltpu.async_copy(*ha, add=True)
            pltpu.async_copy(*hb, add=True)

        def drain_scat(b):
            ha, hb = scat_halves(b)
            pltpu.make_async_copy(*ha).wait()
            pltpu.make_async_copy(*hb).wait()

        def issue_idx(i, b):
            off = ebase + i * CHUNK
            if mode == "split":
                pltpu.async_copy(src.at[pl.ds(off, CHUNK)], idx_v[b], sem_i[b])
                pltpu.async_copy(dst.at[pl.ds(off, CHUNK)], dst_v[b], sem_i[b])
            else:
                pltpu.async_copy(src.at[c, pl.ds(off, CHUNK)], idx_v[b], sem_i[b])
                pltpu.async_copy(dst.at[c, pl.ds(off, CHUNK)], dst_v[b], sem_i[b])

        def drain_idx(b):
            pltpu.make_async_copy(src.at[pl.ds(0, CHUNK)] if mode == "split"
                                  else src.at[0, pl.ds(0, CHUNK)],
                                  idx_v[b], sem_i[b]).wait()
            pltpu.make_async_copy(dst.at[pl.ds(0, CHUNK)] if mode == "split"
                                  else dst.at[0, pl.ds(0, CHUNK)],
                                  dst_v[b], sem_i[b]).wait()

        issue_idx(0, 0)
        issue_idx(1, 1)

        @pl.loop(0, nch // 2)
        def _chunks(g):
            for b in range(2):
                i = 2 * g + b
                drain_idx(b)

                @pl.when(g > 0)
                def _():
                    drain_scat(b)

                pltpu.async_copy(table.at[idx_v[b]], rows_v[b], sem_g[b])
                for k in range(CHUNK // 16):
                    d16 = dst_v[b][pl.ds(k * 16, 16)]
                    plsc.addupdate_scatter(hist_v, [d16], ones16)
                    dst_s[b][pl.ds(k * 16, 16)] = d16
                pltpu.make_async_copy(table.at[idx_v[b]], rows_v[b],
                                      sem_g[b]).wait()

                issue_scat(b)

                @pl.when(i + 2 < nch)
                def _():
                    issue_idx(i + 2, b)

        for b in range(2):
            drain_scat(b)
        plsc.subcore_barrier()
        def rd(j, b):
            return (acc_sh.at[pl.ds(r0 + j * CHUNK, CHUNK)], rows_v[b],
                    sem_g[b])

        def wr(j, b):
            return (rows_v[b], out_s.at[c, pl.ds(r0 + j * CHUNK, CHUNK)],
                    sem_s[b])

        pltpu.async_copy(*rd(0, 0))
        for j in range(nblk):
            b = j & 1
            pltpu.make_async_copy(*rd(j, b)).wait()
            if j >= 1:
                pltpu.make_async_copy(*wr(j - 1, 1 - b)).wait()
            if j + 1 < nblk:
                pltpu.async_copy(*rd(j + 1, 1 - b))
            pltpu.async_copy(*wr(j, b))
        pltpu.make_async_copy(*wr(nblk - 1, (nblk - 1) & 1)).wait()
        pltpu.sync_copy(hist_v, out_h.at[w])

    return pl.kernel(
        body,
        mesh=mesh,
        compiler_params=pltpu.CompilerParams(needs_layout_passes=False),
        out_type=(
            jax.ShapeDtypeStruct((2, NP, D), jnp.float32),
            jax.ShapeDtypeStruct((32, NP), jnp.float32),
        ),
        scratch_types=[
            pltpu.VMEM((CHUNK,), jnp.int32),
            pltpu.VMEM((CHUNK,), jnp.int32),
            pltpu.VMEM((CHUNK,), jnp.int32),
            pltpu.VMEM((CHUNK,), jnp.int32),
            pltpu.VMEM((CHUNK,), jnp.int32),
            pltpu.VMEM((CHUNK,), jnp.int32),
            pltpu.VMEM((CHUNK, D), jnp.float32),
            pltpu.VMEM((CHUNK, D), jnp.float32),
            pltpu.VMEM((NP,), jnp.float32),
            pltpu.VMEM_SHARED((NP, D), jnp.float32),
            pltpu.SemaphoreType.DMA,
            pltpu.SemaphoreType.DMA,
            pltpu.SemaphoreType.DMA,
            pltpu.SemaphoreType.DMA,
            pltpu.SemaphoreType.DMA,
            pltpu.SemaphoreType.DMA,
        ],
    )


_seg_split = _make_seg_kernel("split")
_seg_two = _make_seg_kernel("two")


def _dense1_body(x_ref, s_ref, h_ref, wself_ref, wnbr_ref, be_ref,
                 wg1_ref, bg1_ref, wg2_ref, bg2_ref, he_ref, gh_ref):
    ssum = s_ref[0] + s_ref[1]
    cnt = jnp.sum(h_ref[...], axis=0)[:, None]
    nbr_mean = ssum / jnp.maximum(cnt, 1.0)
    nbr = jnp.dot(nbr_mean, wnbr_ref[...], preferred_element_type=jnp.float32)
    he = jnp.dot(x_ref[...], wself_ref[...], preferred_element_type=jnp.float32)
    he = jnp.maximum(he + nbr + be_ref[...], 0.0)
    g1 = jnp.dot(he, wg1_ref[...], preferred_element_type=jnp.float32)
    g1 = jnp.maximum(g1 + bg1_ref[...], 0.0)
    z = jnp.dot(g1, wg2_ref[...], preferred_element_type=jnp.float32) + bg2_ref[...]
    gate = jax.nn.sigmoid(z)
    he_ref[...] = he
    gh_ref[...] = gate * he


def _dense2_body(he_ref, s_ref, h_ref, wh1_ref, wh2_ref, bf_ref,
                 woa_ref, wob_ref, bo_ref, out_ref):
    c1 = jnp.sum(h_ref[0:16], axis=0)[:, None]
    c2 = jnp.sum(h_ref[16:32], axis=0)[:, None]
    a1 = s_ref[0] / jnp.maximum(c1, 1.0)
    a2 = s_ref[1] / jnp.maximum(c2, 1.0)
    fog = (jnp.dot(a1, wh1_ref[...], preferred_element_type=jnp.float32)
           + jnp.dot(a2, wh2_ref[...], preferred_element_type=jnp.float32))
    hf = jnp.maximum(fog + bf_ref[...], 0.0)
    out_ref[...] = (jnp.dot(he_ref[...], woa_ref[...], preferred_element_type=jnp.float32)
                    + jnp.dot(hf, wob_ref[...], preferred_element_type=jnp.float32)
                    + bo_ref[...])


def _full(shape):
    return pl.BlockSpec(shape, lambda i: tuple(0 for _ in shape))


def _dense1(xp, sums0, hist0, W_self, W_nbr, be, W_g1, bg1, W_g2, bg2):
    return pl.pallas_call(
        _dense1_body,
        grid=(GRID,),
        in_specs=[
            pl.BlockSpec((BR, D), lambda i: (i, 0)),
            pl.BlockSpec((2, BR, D), lambda i: (0, i, 0)),
            pl.BlockSpec((32, BR), lambda i: (0, i)),
            _full((D, D)),
            _full((D, D)),
            _full((1, D)),
            _full((D, 32)),
            _full((1, 32)),
            _full((32, 1)),
            _full((1, 1)),
        ],
        out_specs=[
            pl.BlockSpec((BR, D), lambda i: (i, 0)),
            pl.BlockSpec((BR, D), lambda i: (i, 0)),
        ],
        out_shape=[
            jax.ShapeDtypeStruct((NP, D), jnp.float32),
            jax.ShapeDtypeStruct((NP, D), jnp.float32),
        ],
    )(xp, sums0, hist0, W_self, W_nbr, be, W_g1, bg1, W_g2, bg2)


def _dense2(he, sums12, hist12, W_hop1, W_hop2, bf, woa, wob, bo):
    nc = woa.shape[1]
    return pl.pallas_call(
        _dense2_body,
        grid=(GRID,),
        in_specs=[
            pl.BlockSpec((BR, D), lambda i: (i, 0)),
            pl.BlockSpec((2, BR, D), lambda i: (0, i, 0)),
            pl.BlockSpec((32, BR), lambda i: (0, i)),
            _full((D, D)),
            _full((D, D)),
            _full((1, D)),
            _full((D, nc)),
            _full((D, nc)),
            _full((1, nc)),
        ],
        out_specs=pl.BlockSpec((BR, nc), lambda i: (i, 0)),
        out_shape=jax.ShapeDtypeStruct((NP, nc), jnp.float32),
    )(he, sums12, hist12, W_hop1, W_hop2, bf, woa, wob, bo)


def kernel(x, edge_index_0, edge_index_1, edge_index_2, W_self, W_nbr, b_edge,
           W_g1, b_g1, W_g2, b_g2, W_hop1, W_hop2, b_fog, W_out, b_out):
    E = edge_index_0.shape[1]
    pad = EPAD - E
    xp = jnp.pad(x, ((0, NP - N), (0, 0)))

    def pad_edges(ei):
        src = jnp.concatenate([ei[0], jnp.zeros((pad,), jnp.int32)])
        dst = jnp.concatenate([ei[1], jnp.full((pad,), N, jnp.int32)])
        return src, dst

    s0, d0 = pad_edges(edge_index_0)
    s1, d1 = pad_edges(edge_index_1)
    s2, d2 = pad_edges(edge_index_2)
    src12 = jnp.stack([s1, s2])
    dst12 = jnp.stack([d1, d2])

    zacc = jnp.zeros((CHUNK, D), jnp.float32)
    zhist = jnp.zeros((NP,), jnp.float32)

    sums0, hist0 = _seg_split(xp, s0, d0, zacc, zhist)
    he, gh = _dense1(xp, sums0, hist0, W_self, W_nbr, b_edge.reshape(1, D),
                     W_g1, b_g1.reshape(1, -1), W_g2, b_g2.reshape(1, 1))
    sums12, hist12 = _seg_two(gh, src12, dst12, zacc, zhist)
    logits = _dense2(he, sums12, hist12, W_hop1, W_hop2, b_fog.reshape(1, D),
                     W_out[:D], W_out[D:], b_out.reshape(1, -1))
    return logits[:N]


# speedup vs baseline: 1.0034x; 1.0034x over previous
import jax
import jax.numpy as jnp
from jax import lax
from jax.experimental import pallas as pl
from jax.experimental.pallas import tpu as pltpu
from jax.experimental.pallas import tpu_sc as plsc

N = 10000
D = 128
NP = 10240
CHUNK = 128
EPAD = 327680
NSUB = 16
RPT = NP // NSUB
BR = 1280
GRID = NP // BR


def _make_seg_kernel(mode):
    ept = EPAD // 32 if mode == "split" else EPAD // 16
    nch = ept // CHUNK
    mesh = plsc.VectorSubcoreMesh(core_axis_name="c", subcore_axis_name="s")

    def body(table, src, dst, zacc, zhist, out_s, out_h,
             idx0, idx1, dv0, dv1, ds0, ds1, rv0, rv1, hist_v, acc_sh,
             si0, si1, sg0, sg1, ss0, ss1):
        idx_v = (idx0, idx1)
        dst_v = (dv0, dv1)
        dst_s = (ds0, ds1)
        rows_v = (rv0, rv1)
        sem_i = (si0, si1)
        sem_g = (sg0, sg1)
        sem_s = (ss0, ss1)
        c = lax.axis_index("c")
        s = lax.axis_index("s")
        w = c * NSUB + s
        r0 = s * RPT
        nblk = RPT // CHUNK
        pltpu.sync_copy(zacc, rv0)
        for j in range(nblk):
            pltpu.sync_copy(rv0, acc_sh.at[pl.ds(r0 + j * CHUNK, CHUNK)])
        pltpu.sync_copy(zhist, hist_v)
        plsc.subcore_barrier()

        if mode == "split":
            ebase = (c * NSUB + s) * ept
        else:
            ebase = s * ept

        ones16 = jnp.ones((16,), jnp.float32)

        def issue_idx(i, b):
            off = ebase + i * CHUNK
            if mode == "split":
                pltpu.async_copy(src.at[pl.ds(off, CHUNK)], idx_v[b], sem_i[b])
                pltpu.async_copy(dst.at[pl.ds(off, CHUNK)], dst_v[b], sem_i[b])
            else:
                pltpu.async_copy(src.at[c, pl.ds(off, CHUNK)], idx_v[b], sem_i[b])
                pltpu.async_copy(dst.at[c, pl.ds(off, CHUNK)], dst_v[b], sem_i[b])

        def drain_idx(b):
            pltpu.make_async_copy(src.at[pl.ds(0, CHUNK)] if mode == "split"
                                  else src.at[0, pl.ds(0, CHUNK)],
                                  idx_v[b], sem_i[b]).wait()
            pltpu.make_async_copy(dst.at[pl.ds(0, CHUNK)] if mode == "split"
                                  else dst.at[0, pl.ds(0, CHUNK)],
                                  dst_v[b], sem_i[b]).wait()

        issue_idx(0, 0)
        issue_idx(1, 1)

        @pl.loop(0, nch // 2)
        def _chunks(g):
            for b in range(2):
                i = 2 * g + b
                drain_idx(b)

                @pl.when(g > 0)
                def _():
                    pltpu.make_async_copy(rows_v[b], acc_sh.at[dst_s[b]],
                                          sem_s[b]).wait()

                pltpu.async_copy(table.at[idx_v[b]], rows_v[b], sem_g[b])
                for k in range(CHUNK // 16):
                    d16 = dst_v[b][pl.ds(k * 16, 16)]
                    plsc.addupdate_scatter(hist_v, [d16], ones16)
                    dst_s[b][pl.ds(k * 16, 16)] = d16
                pltpu.make_async_copy(table.at[idx_v[b]], rows_v[b],
                                      sem_g[b]).wait()

                pltpu.async_copy(rows_v[b], acc_sh.at[dst_s[b]], sem_s[b],
                                 add=True)

                @pl.when(i + 2 < nch)
                def _():
                    issue_idx(i + 2, b)

        for b in range(2):
            pltpu.make_async_copy(rows_v[b], acc_sh.at[dst_s[b]],
                                  sem_s[b]).wait()
        plsc.subcore_barrier()
        def rd(j, b):
            return (acc_sh.at[pl.ds(r0 + j * CHUNK, CHUNK)], rows_v[b],
                    sem_g[b])

        def wr(j, b):
            return (rows_v[b], out_s.at[c, pl.ds(r0 + j * CHUNK, CHUNK)],
                    sem_s[b])

        pltpu.async_copy(*rd(0, 0))
        for j in range(nblk):
            b = j & 1
            pltpu.make_async_copy(*rd(j, b)).wait()
            if j >= 1:
                pltpu.make_async_copy(*wr(j - 1, 1 - b)).wait()
            if j + 1 < nblk:
                pltpu.async_copy(*rd(j + 1, 1 - b))
            pltpu.async_copy(*wr(j, b))
        pltpu.make_async_copy(*wr(nblk - 1, (nblk - 1) & 1)).wait()
        pltpu.sync_copy(hist_v, out_h.at[w])

    return pl.kernel(
        body,
        mesh=mesh,
        compiler_params=pltpu.CompilerParams(needs_layout_passes=False),
        out_type=(
            jax.ShapeDtypeStruct((2, NP, D), jnp.float32),
            jax.ShapeDtypeStruct((32, NP), jnp.float32),
        ),
        scratch_types=[
            pltpu.VMEM((CHUNK,), jnp.int32),
            pltpu.VMEM((CHUNK,), jnp.int32),
            pltpu.VMEM((CHUNK,), jnp.int32),
            pltpu.VMEM((CHUNK,), jnp.int32),
            pltpu.VMEM((CHUNK,), jnp.int32),
            pltpu.VMEM((CHUNK,), jnp.int32),
            pltpu.VMEM((CHUNK, D), jnp.float32),
            pltpu.VMEM((CHUNK, D), jnp.float32),
            pltpu.VMEM((NP,), jnp.float32),
            pltpu.VMEM_SHARED((NP, D), jnp.float32),
            pltpu.SemaphoreType.DMA,
            pltpu.SemaphoreType.DMA,
            pltpu.SemaphoreType.DMA,
            pltpu.SemaphoreType.DMA,
            pltpu.SemaphoreType.DMA,
            pltpu.SemaphoreType.DMA,
        ],
    )


_seg_split = _make_seg_kernel("split")
_seg_two = _make_seg_kernel("two")


def _dense1_body(x_ref, s_ref, h_ref, wself_ref, wnbr_ref, be_ref,
                 wg1_ref, bg1_ref, wg2_ref, bg2_ref, he_ref, gh_ref):
    ssum = s_ref[0] + s_ref[1]
    cnt = jnp.sum(h_ref[...], axis=0)[:, None]
    nbr_mean = ssum / jnp.maximum(cnt, 1.0)
    nbr = jnp.dot(nbr_mean, wnbr_ref[...], preferred_element_type=jnp.float32)
    he = jnp.dot(x_ref[...], wself_ref[...], preferred_element_type=jnp.float32)
    he = jnp.maximum(he + nbr + be_ref[...], 0.0)
    g1 = jnp.dot(he, wg1_ref[...], preferred_element_type=jnp.float32)
    g1 = jnp.maximum(g1 + bg1_ref[...], 0.0)
    z = jnp.dot(g1, wg2_ref[...], preferred_element_type=jnp.float32) + bg2_ref[...]
    gate = jax.nn.sigmoid(z)
    he_ref[...] = he
    gh_ref[...] = gate * he


def _dense2_body(he_ref, s_ref, h_ref, wh1_ref, wh2_ref, bf_ref,
                 woa_ref, wob_ref, bo_ref, out_ref):
    c1 = jnp.sum(h_ref[0:16], axis=0)[:, None]
    c2 = jnp.sum(h_ref[16:32], axis=0)[:, None]
    a1 = s_ref[0] / jnp.maximum(c1, 1.0)
    a2 = s_ref[1] / jnp.maximum(c2, 1.0)
    fog = (jnp.dot(a1, wh1_ref[...], preferred_element_type=jnp.float32)
           + jnp.dot(a2, wh2_ref[...], preferred_element_type=jnp.float32))
    hf = jnp.maximum(fog + bf_ref[...], 0.0)
    out_ref[...] = (jnp.dot(he_ref[...], woa_ref[...], preferred_element_type=jnp.float32)
                    + jnp.dot(hf, wob_ref[...], preferred_element_type=jnp.float32)
                    + bo_ref[...])


def _full(shape):
    return pl.BlockSpec(shape, lambda i: tuple(0 for _ in shape))


def _dense1(xp, sums0, hist0, W_self, W_nbr, be, W_g1, bg1, W_g2, bg2):
    return pl.pallas_call(
        _dense1_body,
        grid=(GRID,),
        in_specs=[
            pl.BlockSpec((BR, D), lambda i: (i, 0)),
            pl.BlockSpec((2, BR, D), lambda i: (0, i, 0)),
            pl.BlockSpec((32, BR), lambda i: (0, i)),
            _full((D, D)),
            _full((D, D)),
            _full((1, D)),
            _full((D, 32)),
            _full((1, 32)),
            _full((32, 1)),
            _full((1, 1)),
        ],
        out_specs=[
            pl.BlockSpec((BR, D), lambda i: (i, 0)),
            pl.BlockSpec((BR, D), lambda i: (i, 0)),
        ],
        out_shape=[
            jax.ShapeDtypeStruct((NP, D), jnp.float32),
            jax.ShapeDtypeStruct((NP, D), jnp.float32),
        ],
    )(xp, sums0, hist0, W_self, W_nbr, be, W_g1, bg1, W_g2, bg2)


def _dense2(he, sums12, hist12, W_hop1, W_hop2, bf, woa, wob, bo):
    nc = woa.shape[1]
    return pl.pallas_call(
        _dense2_body,
        grid=(GRID,),
        in_specs=[
            pl.BlockSpec((BR, D), lambda i: (i, 0)),
            pl.BlockSpec((2, BR, D), lambda i: (0, i, 0)),
            pl.BlockSpec((32, BR), lambda i: (0, i)),
            _full((D, D)),
            _full((D, D)),
            _full((1, D)),
            _full((D, nc)),
            _full((D, nc)),
            _full((1, nc)),
        ],
        out_specs=pl.BlockSpec((BR, nc), lambda i: (i, 0)),
        out_shape=jax.ShapeDtypeStruct((NP, nc), jnp.float32),
    )(he, sums12, hist12, W_hop1, W_hop2, bf, woa, wob, bo)


def kernel(x, edge_index_0, edge_index_1, edge_index_2, W_self, W_nbr, b_edge,
           W_g1, b_g1, W_g2, b_g2, W_hop1, W_hop2, b_fog, W_out, b_out):
    E = edge_index_0.shape[1]
    pad = EPAD - E
    xp = jnp.pad(x, ((0, NP - N), (0, 0)))

    def pad_edges(ei):
        src = jnp.concatenate([ei[0], jnp.zeros((pad,), jnp.int32)])
        dst = jnp.concatenate([ei[1], jnp.full((pad,), N, jnp.int32)])
        return src, dst

    s0, d0 = pad_edges(edge_index_0)
    s1, d1 = pad_edges(edge_index_1)
    s2, d2 = pad_edges(edge_index_2)
    src12 = jnp.stack([s1, s2])
    dst12 = jnp.stack([d1, d2])

    zacc = jnp.zeros((CHUNK, D), jnp.float32)
    zhist = jnp.zeros((NP,), jnp.float32)

    sums0, hist0 = _seg_split(xp, s0, d0, zacc, zhist)
    he, gh = _dense1(xp, sums0, hist0, W_self, W_nbr, b_edge.reshape(1, D),
                     W_g1, b_g1.reshape(1, -1), W_g2, b_g2.reshape(1, 1))
    sums12, hist12 = _seg_two(gh, src12, dst12, zacc, zhist)
    logits = _dense2(he, sums12, hist12, W_hop1, W_hop2, b_fog.reshape(1, D),
                     W_out[:D], W_out[D:], b_out.reshape(1, -1))
    return logits[:N]
